# baseline (speedup 1.0000x reference)
import functools
import math

import jax
import jax.numpy as jnp
from jax import lax
from jax.experimental import pallas as pl
from jax.experimental.pallas import tpu as pltpu
from jax.experimental.pallas import tpu_sc as plsc

VOCAB = 100000
D = 128
B_TOKENS = 1024 * 200
NUM_WORKERS = 32
PER_WORKER = B_TOKENS // NUM_WORKERS
CHUNK = 128
CHUNKS_PER_WORKER = PER_WORKER // CHUNK
SCALE = math.sqrt(D)

_mesh = plsc.VectorSubcoreMesh(core_axis_name="c", subcore_axis_name="s")

NBUF = 5
STEADY_GROUPS = CHUNKS_PER_WORKER // NBUF


@functools.partial(
    pl.kernel,
    out_type=jax.ShapeDtypeStruct((B_TOKENS, D), jnp.float32),
    mesh=_mesh,
    scratch_types=[
        pltpu.VMEM((CHUNKS_PER_WORKER, CHUNK), jnp.int32),
        [pltpu.VMEM((CHUNK, D), jnp.float32) for _ in range(NBUF)],
        pltpu.SemaphoreType.DMA((NBUF,)),
        pltpu.SemaphoreType.DMA((NBUF,)),
    ],
)
def _embed_gather(table_hbm, tok_hbm, out_hbm, idx_v, bufs, gsems, wsems):
    wid = lax.axis_index("s") * 2 + lax.axis_index("c")
    base = wid * PER_WORKER
    pltpu.sync_copy(tok_hbm.at[wid], idx_v)

    def gather_start(j, k):
        pltpu.async_copy(table_hbm.at[idx_v.at[j]], bufs[k], gsems.at[k])

    def gather_wait(j, k):
        pltpu.make_async_copy(table_hbm.at[idx_v.at[j]], bufs[k],
                              gsems.at[k]).wait()

    def write_start(j, k):
        pltpu.async_copy(bufs[k], out_hbm.at[pl.ds(base + j * CHUNK, CHUNK)],
                         wsems.at[k])

    def write_wait(j, k):
        pltpu.make_async_copy(bufs[k],
                              out_hbm.at[pl.ds(base + j * CHUNK, CHUNK)],
                              wsems.at[k]).wait()

    def scale_buf(k):
        buf = bufs[k]

        @plsc.parallel_loop(0, CHUNK, unroll=8)
        def _(r):
            for c in range(D // 16):
                buf[r, pl.ds(c * 16, 16)] = buf[r, pl.ds(c * 16, 16)] * SCALE

    def step(j, k, wait_write, prefetch):
        gather_wait(j, k)
        scale_buf(k)
        write_start(j, k)
        kb = (k + NBUF - 1) % NBUF
        if prefetch:
            if wait_write:
                write_wait(j - 1, kb)
            gather_start(j + NBUF - 1, kb)

    for k in range(NBUF - 1):
        gather_start(k, k)
    for k in range(NBUF):
        step(k, k, wait_write=(k >= 1), prefetch=True)

    def group_body(g, carry):
        for k in range(NBUF):
            step(NBUF * g + k, k, wait_write=True, prefetch=True)
        return carry

    lax.fori_loop(1, STEADY_GROUPS - 1, group_body, 0)

    j0 = NBUF * (STEADY_GROUPS - 1)
    step(j0, 0, wait_write=True, prefetch=True)
    for k in range(1, NBUF):
        step(j0 + k, k, wait_write=False, prefetch=False)
    for k in range(NBUF):
        write_wait(j0 + k, k)


def kernel(tokens, table):
    tok = tokens.reshape(NUM_WORKERS, CHUNKS_PER_WORKER, CHUNK).astype(jnp.int32)
    out = _embed_gather(table, tok)
    return out.reshape(tokens.shape[0], tokens.shape[1], D)


# speedup vs baseline: 1.0213x; 1.0213x over previous
import functools
import math

import jax
import jax.numpy as jnp
from jax import lax
from jax.experimental import pallas as pl
from jax.experimental.pallas import tpu as pltpu
from jax.experimental.pallas import tpu_sc as plsc

VOCAB = 100000
D = 128
B_TOKENS = 1024 * 200
NUM_WORKERS = 32
PER_WORKER = B_TOKENS // NUM_WORKERS
CHUNK = 128
CHUNKS_PER_WORKER = PER_WORKER // CHUNK
SCALE = math.sqrt(D)

_mesh = plsc.VectorSubcoreMesh(core_axis_name="c", subcore_axis_name="s")

NBUF = 5
STEADY_GROUPS = CHUNKS_PER_WORKER // NBUF


@functools.partial(
    pl.kernel,
    out_type=jax.ShapeDtypeStruct((B_TOKENS, D), jnp.float32),
    mesh=_mesh,
    scratch_types=[
        pltpu.VMEM((CHUNKS_PER_WORKER, CHUNK), jnp.int32),
        [pltpu.VMEM((CHUNK, D), jnp.float32) for _ in range(NBUF)],
        pltpu.SemaphoreType.DMA((NBUF,)),
        pltpu.SemaphoreType.DMA((NBUF,)),
    ],
)
def _embed_gather(table_hbm, tok_hbm, out_hbm, idx_v, bufs, gsems, wsems):
    wid = lax.axis_index("s") * 2 + lax.axis_index("c")
    base = wid * PER_WORKER
    pltpu.sync_copy(tok_hbm.at[wid], idx_v)

    def gather_start(j, k):
        pltpu.async_copy(table_hbm.at[idx_v.at[j]], bufs[k], gsems.at[k])

    def gather_wait(j, k):
        pltpu.make_async_copy(table_hbm.at[idx_v.at[j]], bufs[k],
                              gsems.at[k]).wait()

    def write_start(j, k):
        pltpu.async_copy(bufs[k], out_hbm.at[pl.ds(base + j * CHUNK, CHUNK)],
                         wsems.at[k])

    def write_wait(j, k):
        pltpu.make_async_copy(bufs[k],
                              out_hbm.at[pl.ds(base + j * CHUNK, CHUNK)],
                              wsems.at[k]).wait()

    def scale_buf(k):
        buf = bufs[k]

        @plsc.parallel_loop(0, CHUNK, unroll=4)
        def _(r):
            for c in range(D // 16):
                buf[r, pl.ds(c * 16, 16)] = buf[r, pl.ds(c * 16, 16)] * SCALE

    def step(j, k, wait_write, prefetch):
        gather_wait(j, k)
        scale_buf(k)
        write_start(j, k)
        kb = (k + NBUF - 1) % NBUF
        if prefetch:
            if wait_write:
                write_wait(j - 1, kb)
            gather_start(j + NBUF - 1, kb)

    for k in range(NBUF - 1):
        gather_start(k, k)
    for k in range(NBUF):
        step(k, k, wait_write=(k >= 1), prefetch=True)

    def group_body(g, carry):
        for k in range(NBUF):
            step(NBUF * g + k, k, wait_write=True, prefetch=True)
        return carry

    lax.fori_loop(1, STEADY_GROUPS - 1, group_body, 0)

    j0 = NBUF * (STEADY_GROUPS - 1)
    step(j0, 0, wait_write=True, prefetch=True)
    for k in range(1, NBUF):
        step(j0 + k, k, wait_write=False, prefetch=False)
    for k in range(NBUF):
        write_wait(j0 + k, k)


def kernel(tokens, table):
    tok = tokens.reshape(NUM_WORKERS, CHUNKS_PER_WORKER, CHUNK).astype(jnp.int32)
    out = _embed_gather(table, tok)
    return out.reshape(tokens.shape[0], tokens.shape[1], D)
